# baseline (speedup 1.0000x reference)
import functools

import jax
import jax.numpy as jnp
from jax import lax
from jax.experimental import pallas as pl
from jax.experimental.pallas import tpu as pltpu
from jax.experimental.pallas import tpu_sc as plsc

H = 768
E = 8
T = 4
TD = 64
K = 2
I = 1536
S = 2048
AD = TD * T
HP = 4 * H

TB_A = 256
NTB_A = S // TB_A

NA = K * S
BLKS = 256
NB = NA // BLKS + E
C = NB * BLKS

NBPAD = ((NB + 15) // 16) * 16

NC = 2
NS = 16
NW = NC * NS
ACH = NA // NW
TCH = S // NW


def _sigmoid(x):
    return 1.0 / (1.0 + jnp.exp(-x))


def _router_kernel(x_ref, te_ref, wip_ref, bip_ref, wmp_ref, bmp_ref,
                   wr_ref, br_ref, pattr_ref, attr_ref, wap_ref, bap_ref,
                   wg1a_ref, bg1_ref, pm_ref, ent_ref,
                   i1_ref, i2_ref, p1_ref, p2_ref, c1_ref, c2_ref,
                   apo_ref, cgi_ref):
    ap = attr_ref[0] @ wap_ref[0] + bap_ref[0]
    apo_ref[0] = ap
    cgi_ref[0] = ap @ wg1a_ref[0] + bg1_ref[0]
    xs = x_ref[...]
    te = te_ref[...]
    h = xs @ wip_ref[:H, :] + te @ wip_ref[H:, :] + bip_ref[...]
    h = jnp.maximum(h, 0.0)
    h = h @ wmp_ref[...] + bmp_ref[...]
    h = jnp.maximum(h, 0.0)
    logits = h @ wr_ref[...] + br_ref[...]
    logits = logits - jnp.max(logits, axis=-1, keepdims=True)
    el = jnp.exp(logits)
    ep = el / jnp.sum(el, axis=-1, keepdims=True)
    acc = jnp.zeros((TB_A, E), jnp.float32)
    for t in range(T):
        asc = te[:, t * TD:(t + 1) * TD] @ pattr_ref[...]
        asc = asc - jnp.max(asc, axis=-1, keepdims=True)
        ea = jnp.exp(asc)
        acc = acc + ea / jnp.sum(ea, axis=-1, keepdims=True)
    p = ep * (acc * (1.0 / T))
    iota8 = lax.broadcasted_iota(jnp.int32, (TB_A, E), 1)
    m1 = jnp.max(p, axis=-1, keepdims=True)
    i1 = jnp.min(jnp.where(p == m1, iota8, E), axis=-1, keepdims=True)
    px = jnp.where(iota8 == i1, -1.0, p)
    m2 = jnp.max(px, axis=-1, keepdims=True)
    i2 = jnp.min(jnp.where(px == m2, iota8, E), axis=-1, keepdims=True)
    mask1 = iota8 == i1
    mask2 = iota8 == i2
    pm = p * jnp.logical_or(mask1, mask2).astype(jnp.float32)
    pm_ref[...] = pm
    ent = jnp.sum(pm * jnp.log(pm + 1e-08))
    ent_ref[...] = jnp.full((1, 8, 128), ent, jnp.float32)
    i1_ref[...] = i1
    i2_ref[...] = i2
    p1_ref[...] = m1
    p2_ref[...] = m2
    m1i = mask1.astype(jnp.int32)
    m2i = mask2.astype(jnp.int32)
    z8 = jnp.zeros((1, 8), jnp.int32)
    c1_ref[0] = jnp.concatenate(
        [jnp.sum(m1i[:128], axis=0, keepdims=True), z8,
         jnp.sum(m1i[128:], axis=0, keepdims=True), z8], axis=1).reshape(2, 16)
    c2_ref[0] = jnp.concatenate(
        [jnp.sum(m2i[:128], axis=0, keepdims=True), z8,
         jnp.sum(m2i[128:], axis=0, keepdims=True), z8], axis=1).reshape(2, 16)


def _dispatch_kernel(i1_hbm, i2_hbm, c1_hbm, c2_hbm, x_hbm,
                     xd_hbm, dpos_hbm, be_hbm, used_hbm,
                     ids_v, cnts_v, tbuf_v, dbuf_v, bebuf_v, used_v,
                     rows_v, sem):
    wid = lax.axis_index("s") * NC + lax.axis_index("c")
    iota = jnp.arange(16, dtype=jnp.int32)

    @pl.when(wid < NW // 2)
    def _():
        pltpu.sync_copy(i1_hbm.at[pl.ds(wid * ACH, ACH)], ids_v)

    @pl.when(wid >= NW // 2)
    def _():
        pltpu.sync_copy(i2_hbm.at[pl.ds(wid * ACH - S, ACH)], ids_v)

    pltpu.sync_copy(c1_hbm, cnts_v.at[pl.ds(0, NW * 8)])
    pltpu.sync_copy(c2_hbm, cnts_v.at[pl.ds(NW * 8, NW * 8)])

    def crow(r, carry):
        tot, pre = carry
        row = cnts_v[pl.ds(r * 16, 16)]
        pre = jnp.where(r < wid, pre + row, pre)
        tot = tot + row
        return tot, pre

    zero16 = jnp.zeros(16, jnp.int32)
    tot, pre = lax.fori_loop(0, NW, crow, (zero16, zero16))

    npad = jnp.where(iota < E, (tot + (BLKS - 1)) & (-BLKS), 0)
    ends = plsc.cumsum(npad)
    breg = ends - npad + pre

    for j in range(ACH // 16):
        ids = ids_v[pl.ds(j * 16, 16)]
        av = iota + (wid * ACH + j * 16)
        tbuf_v[pl.ds(j * 16, 16)] = av & (S - 1)
        dest = zero16
        for e in range(E):
            m = ids == e
            r = plsc.cumsum(m.astype(jnp.int32))
            bsp = jnp.sum(jnp.where(iota == e, breg, 0))
            dest = jnp.where(m, r - 1 + bsp, dest)
            pc = plsc.all_reduce_population_count(m)
            breg = breg + jnp.where(iota == e, pc, 0)
        dbuf_v[pl.ds(j * 16, 16)] = dest

    cp_g = pltpu.async_copy(x_hbm.at[tbuf_v], rows_v, sem)
    cp_g.wait()
    cp_s = pltpu.async_copy(rows_v, xd_hbm.at[dbuf_v], sem)
    cp_s.wait()
    pltpu.sync_copy(dbuf_v, dpos_hbm.at[pl.ds(wid * ACH, ACH)])

    @pl.when(wid == 0)
    def _():
        for vi in range(NBPAD // 16):
            bv = (iota + vi * 16) * BLKS
            acc = jnp.zeros(16, jnp.int32)
            for e in range(E):
                ee = jnp.sum(jnp.where(iota == e, ends, 0))
                acc = acc + (bv >= ee).astype(jnp.int32)
            bebuf_v[pl.ds(vi * 16, 16)] = jnp.minimum(acc, E - 1)
        pltpu.sync_copy(bebuf_v, be_hbm)
        used_v[...] = jnp.sum(jnp.where(iota == E - 1, ends, 0)) + iota * 0
        pltpu.sync_copy(used_v, used_hbm)


def _group_ffn_kernel(be_ref, used_ref, ap_ref, cgi_ref, wg1x_ref, wg2_ref,
                      bg2_ref, wf1_ref, bf1_ref, wf2_ref, bf2_ref, lng_ref,
                      lnb_ref, xd_ref, yd_ref, x2_scr):
    b = pl.program_id(0)
    s = pl.program_id(1)
    live = b * BLKS < used_ref[0]

    @pl.when(jnp.logical_and(live, s == 0))
    def _():
        ap = ap_ref[0]
        xs = xd_ref[...]
        gi = xs @ wg1x_ref[0] + cgi_ref[0]
        g = jnp.maximum(gi, 0.0) @ wg2_ref[0] + bg2_ref[0]
        g = _sigmoid(g)
        x2_scr[...] = xs * g + ap * (1.0 - g)

    @pl.when(jnp.logical_and(live, s == 1))
    def _():
        xs = xd_ref[...]
        x2 = x2_scr[...]
        x3 = jnp.maximum(x2 @ wf1_ref[0] + bf1_ref[0], 0.0)
        y = x3 @ wf2_ref[0] + bf2_ref[0] + xs
        m = jnp.mean(y, axis=-1, keepdims=True)
        yc = y - m
        v = jnp.mean(yc * yc, axis=-1, keepdims=True)
        yd_ref[...] = yc / jnp.sqrt(v + 1e-05) * lng_ref[0] + lnb_ref[0]


def _combine_kernel(yd_hbm, dpos_hbm, p1_hbm, p2_hbm, out_hbm,
                    idx0_v, idx1_v, p1_v, p2_v, buf0_v, buf1_v, sem):
    wid = lax.axis_index("s") * NC + lax.axis_index("c")
    base_t = wid * TCH
    pltpu.sync_copy(dpos_hbm.at[pl.ds(base_t, TCH)], idx0_v)
    pltpu.sync_copy(dpos_hbm.at[pl.ds(S + base_t, TCH)], idx1_v)
    pltpu.sync_copy(p1_hbm.at[pl.ds(base_t, TCH)], p1_v)
    pltpu.sync_copy(p2_hbm.at[pl.ds(base_t, TCH)], p2_v)
    cp0 = pltpu.async_copy(yd_hbm.at[idx0_v], buf0_v, sem)
    cp1 = pltpu.async_copy(yd_hbm.at[idx1_v], buf1_v, sem)
    cp0.wait()
    cp1.wait()

    def row(i, _):
        fi = jnp.full((16,), i, jnp.int32)
        p1b = plsc.load_gather(p1_v, [fi])
        p2b = plsc.load_gather(p2_v, [fi])
        for ch in range(H // 16):
            sl = pl.ds(ch * 16, 16)
            buf0_v[i, sl] = buf0_v[i, sl] * p1b + buf1_v[i, sl] * p2b
        return 0

    lax.fori_loop(0, TCH, row, 0)
    pltpu.sync_copy(buf0_v, out_hbm.at[pl.ds(base_t, TCH)])


_sc_mesh = plsc.VectorSubcoreMesh(
    core_axis_name="c", subcore_axis_name="s", num_cores=NC, num_subcores=NS)

_sc_params = pltpu.CompilerParams(needs_layout_passes=False)

_dispatch = functools.partial(
    pl.kernel,
    compiler_params=_sc_params,
    out_type=[
        jax.ShapeDtypeStruct((C, H), jnp.float32),
        jax.ShapeDtypeStruct((NA,), jnp.int32),
        jax.ShapeDtypeStruct((NBPAD,), jnp.int32),
        jax.ShapeDtypeStruct((16,), jnp.int32),
    ],
    mesh=_sc_mesh,
    scratch_types=[
        pltpu.VMEM((ACH,), jnp.int32),
        pltpu.VMEM((NW * 16,), jnp.int32),
        pltpu.VMEM((ACH,), jnp.int32),
        pltpu.VMEM((ACH,), jnp.int32),
        pltpu.VMEM((NBPAD,), jnp.int32),
        pltpu.VMEM((16,), jnp.int32),
        pltpu.VMEM((ACH, H), jnp.float32),
        pltpu.SemaphoreType.DMA,
    ],
)(_dispatch_kernel)

_combine = functools.partial(
    pl.kernel,
    compiler_params=_sc_params,
    out_type=jax.ShapeDtypeStruct((S, H), jnp.float32),
    mesh=_sc_mesh,
    scratch_types=[
        pltpu.VMEM((TCH,), jnp.int32),
        pltpu.VMEM((TCH,), jnp.int32),
        pltpu.VMEM((TCH,), jnp.float32),
        pltpu.VMEM((TCH,), jnp.float32),
        pltpu.VMEM((TCH, H), jnp.float32),
        pltpu.VMEM((TCH, H), jnp.float32),
        pltpu.SemaphoreType.DMA,
    ],
)(_combine_kernel)


@jax.jit
def _forward(x, task_embeddings, params):
    x2d = x.reshape(S, H)
    te2d = task_embeddings.reshape(S, AD)

    (pm, ent_part, i1, i2, p1, p2, c1, c2, ap_all, cgi_all) = pl.pallas_call(
        _router_kernel,
        grid=(NTB_A,),
        in_specs=[
            pl.BlockSpec((TB_A, H), lambda i: (i, 0)),
            pl.BlockSpec((TB_A, AD), lambda i: (i, 0)),
            pl.BlockSpec((H + AD, HP), lambda i: (0, 0)),
            pl.BlockSpec((1, HP), lambda i: (0, 0)),
            pl.BlockSpec((HP, H), lambda i: (0, 0)),
            pl.BlockSpec((1, H), lambda i: (0, 0)),
            pl.BlockSpec((H, E), lambda i: (0, 0)),
            pl.BlockSpec((1, E), lambda i: (0, 0)),
            pl.BlockSpec((TD, E), lambda i: (0, 0)),
            pl.BlockSpec((1, 1, AD), lambda i: (i, 0, 0)),
            pl.BlockSpec((1, AD, H), lambda i: (i, 0, 0)),
            pl.BlockSpec((1, 1, H), lambda i: (i, 0, 0)),
            pl.BlockSpec((1, H, I), lambda i: (i, 1, 0)),
            pl.BlockSpec((1, 1, I), lambda i: (i, 0, 0)),
        ],
        out_specs=[
            pl.BlockSpec((TB_A, E), lambda i: (i, 0)),
            pl.BlockSpec((1, 8, 128), lambda i: (i, 0, 0)),
            pl.BlockSpec((TB_A, 1), lambda i: (i, 0)),
            pl.BlockSpec((TB_A, 1), lambda i: (i, 0)),
            pl.BlockSpec((TB_A, 1), lambda i: (i, 0)),
            pl.BlockSpec((TB_A, 1), lambda i: (i, 0)),
            pl.BlockSpec((1, 2, 16), lambda i: (i, 0, 0)),
            pl.BlockSpec((1, 2, 16), lambda i: (i, 0, 0)),
            pl.BlockSpec((1, 1, H), lambda i: (i, 0, 0)),
            pl.BlockSpec((1, 1, I), lambda i: (i, 0, 0)),
        ],
        out_shape=[
            jax.ShapeDtypeStruct((S, E), jnp.float32),
            jax.ShapeDtypeStruct((NTB_A, 8, 128), jnp.float32),
            jax.ShapeDtypeStruct((S, 1), jnp.int32),
            jax.ShapeDtypeStruct((S, 1), jnp.int32),
            jax.ShapeDtypeStruct((S, 1), jnp.float32),
            jax.ShapeDtypeStruct((S, 1), jnp.float32),
            jax.ShapeDtypeStruct((NTB_A, 2, 16), jnp.int32),
            jax.ShapeDtypeStruct((NTB_A, 2, 16), jnp.int32),
            jax.ShapeDtypeStruct((E, 1, H), jnp.float32),
            jax.ShapeDtypeStruct((E, 1, I), jnp.float32),
        ],
    )(x2d, te2d, params['Wip'], params['bip'].reshape(1, HP),
      params['Wmp'], params['bmp'].reshape(1, H),
      params['Wr'], params['br'].reshape(1, E), params['P_attr'],
      params['attr_emb'].reshape(E, 1, AD), params['Wap'],
      params['bap'].reshape(E, 1, H), params['Wg1'],
      params['bg1'].reshape(E, 1, I))

    xd, dpos, be, used = _dispatch(i1.reshape(S), i2.reshape(S),
                                   c1.reshape(NW * 8), c2.reshape(NW * 8),
                                   x2d)

    yd = pl.pallas_call(
        _group_ffn_kernel,
        grid_spec=pltpu.PrefetchScalarGridSpec(
            num_scalar_prefetch=2,
            grid=(NB, 2),
            in_specs=[
                pl.BlockSpec((1, 1, H), lambda b, s, be, u: (be[b], 0, 0)),
                pl.BlockSpec((1, 1, I), lambda b, s, be, u: (be[b], 0, 0)),
                pl.BlockSpec((1, H, I), lambda b, s, be, u: (be[b], 0, 0)),
                pl.BlockSpec((1, I, H), lambda b, s, be, u: (be[b], 0, 0)),
                pl.BlockSpec((1, 1, H), lambda b, s, be, u: (be[b], 0, 0)),
                pl.BlockSpec((1, H, I), lambda b, s, be, u: (be[b], 0, 0)),
                pl.BlockSpec((1, 1, I), lambda b, s, be, u: (be[b], 0, 0)),
                pl.BlockSpec((1, I, H), lambda b, s, be, u: (be[b], 0, 0)),
                pl.BlockSpec((1, 1, H), lambda b, s, be, u: (be[b], 0, 0)),
                pl.BlockSpec((1, 1, H), lambda b, s, be, u: (be[b], 0, 0)),
                pl.BlockSpec((1, 1, H), lambda b, s, be, u: (be[b], 0, 0)),
                pl.BlockSpec((BLKS, H), lambda b, s, be, u: (b, 0)),
            ],
            out_specs=pl.BlockSpec((BLKS, H), lambda b, s, be, u: (b, 0)),
            scratch_shapes=[pltpu.VMEM((BLKS, H), jnp.float32)],
        ),
        out_shape=jax.ShapeDtypeStruct((C, H), jnp.float32),
    )(be, used, ap_all, cgi_all, params['Wg1'],
      params['Wg2'], params['bg2'].reshape(E, 1, H),
      params['Wf1'], params['bf1'].reshape(E, 1, I),
      params['Wf2'], params['bf2'].reshape(E, 1, H),
      params['ln_g'].reshape(E, 1, H), params['ln_b'].reshape(E, 1, H),
      xd)

    out = _combine(yd, dpos, p1[:, 0], p2[:, 0])

    entropy_loss = -(jnp.sum(ent_part[:, 0, 0]) / S)
    return out.reshape(x.shape), entropy_loss


def kernel(x, task_embeddings, params):
    return _forward(x, task_embeddings, params)


# speedup vs baseline: 1.1730x; 1.1730x over previous
import functools

import jax
import jax.numpy as jnp
from jax import lax
from jax.experimental import pallas as pl
from jax.experimental.pallas import tpu as pltpu
from jax.experimental.pallas import tpu_sc as plsc

H = 768
E = 8
T = 4
TD = 64
K = 2
I = 1536
S = 2048
AD = TD * T
HP = 4 * H

TB_A = 256
NTB_A = S // TB_A

NA = K * S
BLKS = 512
NB = NA // BLKS + E
C = NB * BLKS

NBPAD = ((NB + 15) // 16) * 16

NC = 2
NS = 16
NW = NC * NS
ACH = NA // NW
TCH = S // NW


def _sigmoid(x):
    return 1.0 / (1.0 + jnp.exp(-x))


def _router_kernel(x_ref, te_ref, wip_ref, bip_ref, wmp_ref, bmp_ref,
                   wr_ref, br_ref, pattr_ref, attr_ref, wap_ref, bap_ref,
                   wg1a_ref, bg1_ref, pm_ref, ent_ref,
                   i1_ref, i2_ref, p1_ref, p2_ref, c1_ref, c2_ref,
                   apo_ref, cgi_ref):
    ap = attr_ref[0] @ wap_ref[0] + bap_ref[0]
    apo_ref[0] = ap
    cgi_ref[0] = ap @ wg1a_ref[0] + bg1_ref[0]
    xs = x_ref[...]
    te = te_ref[...]
    h = xs @ wip_ref[:H, :] + te @ wip_ref[H:, :] + bip_ref[...]
    h = jnp.maximum(h, 0.0)
    h = h @ wmp_ref[...] + bmp_ref[...]
    h = jnp.maximum(h, 0.0)
    logits = h @ wr_ref[...] + br_ref[...]
    logits = logits - jnp.max(logits, axis=-1, keepdims=True)
    el = jnp.exp(logits)
    ep = el / jnp.sum(el, axis=-1, keepdims=True)
    acc = jnp.zeros((TB_A, E), jnp.float32)
    for t in range(T):
        asc = te[:, t * TD:(t + 1) * TD] @ pattr_ref[...]
        asc = asc - jnp.max(asc, axis=-1, keepdims=True)
        ea = jnp.exp(asc)
        acc = acc + ea / jnp.sum(ea, axis=-1, keepdims=True)
    p = ep * (acc * (1.0 / T))
    iota8 = lax.broadcasted_iota(jnp.int32, (TB_A, E), 1)
    m1 = jnp.max(p, axis=-1, keepdims=True)
    i1 = jnp.min(jnp.where(p == m1, iota8, E), axis=-1, keepdims=True)
    px = jnp.where(iota8 == i1, -1.0, p)
    m2 = jnp.max(px, axis=-1, keepdims=True)
    i2 = jnp.min(jnp.where(px == m2, iota8, E), axis=-1, keepdims=True)
    mask1 = iota8 == i1
    mask2 = iota8 == i2
    pm = p * jnp.logical_or(mask1, mask2).astype(jnp.float32)
    pm_ref[...] = pm
    ent = jnp.sum(pm * jnp.log(pm + 1e-08))
    ent_ref[...] = jnp.full((1, 8, 128), ent, jnp.float32)
    i1_ref[...] = i1
    i2_ref[...] = i2
    p1_ref[...] = m1
    p2_ref[...] = m2
    m1i = mask1.astype(jnp.int32)
    m2i = mask2.astype(jnp.int32)
    z8 = jnp.zeros((1, 8), jnp.int32)
    c1_ref[0] = jnp.concatenate(
        [jnp.sum(m1i[:128], axis=0, keepdims=True), z8,
         jnp.sum(m1i[128:], axis=0, keepdims=True), z8], axis=1).reshape(2, 16)
    c2_ref[0] = jnp.concatenate(
        [jnp.sum(m2i[:128], axis=0, keepdims=True), z8,
         jnp.sum(m2i[128:], axis=0, keepdims=True), z8], axis=1).reshape(2, 16)


def _dispatch_kernel(i1_hbm, i2_hbm, c1_hbm, c2_hbm, x_hbm,
                     xd_hbm, dpos_hbm, be_hbm, used_hbm,
                     ids_v, cnts_v, tbuf_v, dbuf_v, bebuf_v, used_v,
                     rows_v, sem):
    wid = lax.axis_index("s") * NC + lax.axis_index("c")
    iota = jnp.arange(16, dtype=jnp.int32)

    @pl.when(wid < NW // 2)
    def _():
        pltpu.sync_copy(i1_hbm.at[pl.ds(wid * ACH, ACH)], ids_v)

    @pl.when(wid >= NW // 2)
    def _():
        pltpu.sync_copy(i2_hbm.at[pl.ds(wid * ACH - S, ACH)], ids_v)

    pltpu.sync_copy(c1_hbm, cnts_v.at[pl.ds(0, NW * 8)])
    pltpu.sync_copy(c2_hbm, cnts_v.at[pl.ds(NW * 8, NW * 8)])

    def crow(r, carry):
        tot, pre = carry
        row = cnts_v[pl.ds(r * 16, 16)]
        pre = jnp.where(r < wid, pre + row, pre)
        tot = tot + row
        return tot, pre

    zero16 = jnp.zeros(16, jnp.int32)
    tot, pre = lax.fori_loop(0, NW, crow, (zero16, zero16))

    npad = jnp.where(iota < E, (tot + (BLKS - 1)) & (-BLKS), 0)
    ends = plsc.cumsum(npad)
    breg = ends - npad + pre

    for j in range(ACH // 16):
        ids = ids_v[pl.ds(j * 16, 16)]
        av = iota + (wid * ACH + j * 16)
        tbuf_v[pl.ds(j * 16, 16)] = av & (S - 1)
        dest = zero16
        for e in range(E):
            m = ids == e
            r = plsc.cumsum(m.astype(jnp.int32))
            bsp = jnp.sum(jnp.where(iota == e, breg, 0))
            dest = jnp.where(m, r - 1 + bsp, dest)
            pc = plsc.all_reduce_population_count(m)
            breg = breg + jnp.where(iota == e, pc, 0)
        dbuf_v[pl.ds(j * 16, 16)] = dest

    cp_g = pltpu.async_copy(x_hbm.at[tbuf_v], rows_v, sem)
    cp_g.wait()
    cp_s = pltpu.async_copy(rows_v, xd_hbm.at[dbuf_v], sem)
    cp_s.wait()
    pltpu.sync_copy(dbuf_v, dpos_hbm.at[pl.ds(wid * ACH, ACH)])

    @pl.when(wid == 0)
    def _():
        for vi in range(NBPAD // 16):
            bv = (iota + vi * 16) * BLKS
            acc = jnp.zeros(16, jnp.int32)
            for e in range(E):
                ee = jnp.sum(jnp.where(iota == e, ends, 0))
                acc = acc + (bv >= ee).astype(jnp.int32)
            bebuf_v[pl.ds(vi * 16, 16)] = jnp.minimum(acc, E - 1)
        pltpu.sync_copy(bebuf_v, be_hbm)
        used_v[...] = jnp.sum(jnp.where(iota == E - 1, ends, 0)) + iota * 0
        pltpu.sync_copy(used_v, used_hbm)


def _group_ffn_kernel(be_ref, used_ref, ap_ref, cgi_ref, wg1x_ref, wg2_ref,
                      bg2_ref, wf1_ref, bf1_ref, wf2_ref, bf2_ref, lng_ref,
                      lnb_ref, xd_ref, yd_ref):
    b = pl.program_id(0)

    @pl.when(b * BLKS < used_ref[0])
    def _():
        _group_ffn_body(ap_ref, cgi_ref, wg1x_ref, wg2_ref, bg2_ref,
                        wf1_ref, bf1_ref, wf2_ref, bf2_ref, lng_ref,
                        lnb_ref, xd_ref, yd_ref)


def _group_ffn_body(ap_ref, cgi_ref, wg1x_ref, wg2_ref, bg2_ref,
                    wf1_ref, bf1_ref, wf2_ref, bf2_ref, lng_ref,
                    lnb_ref, xd_ref, yd_ref):
    ap = ap_ref[0]
    c_gi = cgi_ref[0]
    xs = xd_ref[...]
    gi = xs @ wg1x_ref[0] + c_gi
    g = jnp.maximum(gi, 0.0) @ wg2_ref[0] + bg2_ref[0]
    g = _sigmoid(g)
    x2 = xs * g + ap * (1.0 - g)
    x3 = jnp.maximum(x2 @ wf1_ref[0] + bf1_ref[0], 0.0)
    y = x3 @ wf2_ref[0] + bf2_ref[0] + xs
    m = jnp.mean(y, axis=-1, keepdims=True)
    yc = y - m
    v = jnp.mean(yc * yc, axis=-1, keepdims=True)
    yd_ref[...] = yc / jnp.sqrt(v + 1e-05) * lng_ref[0] + lnb_ref[0]


def _combine_kernel(yd_hbm, dpos_hbm, p1_hbm, p2_hbm, out_hbm,
                    idx0_v, idx1_v, p1_v, p2_v, buf0_v, buf1_v, sem):
    wid = lax.axis_index("s") * NC + lax.axis_index("c")
    base_t = wid * TCH
    pltpu.sync_copy(dpos_hbm.at[pl.ds(base_t, TCH)], idx0_v)
    pltpu.sync_copy(dpos_hbm.at[pl.ds(S + base_t, TCH)], idx1_v)
    pltpu.sync_copy(p1_hbm.at[pl.ds(base_t, TCH)], p1_v)
    pltpu.sync_copy(p2_hbm.at[pl.ds(base_t, TCH)], p2_v)
    cp0 = pltpu.async_copy(yd_hbm.at[idx0_v], buf0_v, sem)
    cp1 = pltpu.async_copy(yd_hbm.at[idx1_v], buf1_v, sem)
    cp0.wait()
    cp1.wait()

    def row(i, _):
        fi = jnp.full((16,), i, jnp.int32)
        p1b = plsc.load_gather(p1_v, [fi])
        p2b = plsc.load_gather(p2_v, [fi])
        for ch in range(H // 16):
            sl = pl.ds(ch * 16, 16)
            buf0_v[i, sl] = buf0_v[i, sl] * p1b + buf1_v[i, sl] * p2b
        return 0

    lax.fori_loop(0, TCH, row, 0)
    pltpu.sync_copy(buf0_v, out_hbm.at[pl.ds(base_t, TCH)])


_sc_mesh = plsc.VectorSubcoreMesh(
    core_axis_name="c", subcore_axis_name="s", num_cores=NC, num_subcores=NS)

_sc_params = pltpu.CompilerParams(needs_layout_passes=False)

_dispatch = functools.partial(
    pl.kernel,
    compiler_params=_sc_params,
    out_type=[
        jax.ShapeDtypeStruct((C, H), jnp.float32),
        jax.ShapeDtypeStruct((NA,), jnp.int32),
        jax.ShapeDtypeStruct((NBPAD,), jnp.int32),
        jax.ShapeDtypeStruct((16,), jnp.int32),
    ],
    mesh=_sc_mesh,
    scratch_types=[
        pltpu.VMEM((ACH,), jnp.int32),
        pltpu.VMEM((NW * 16,), jnp.int32),
        pltpu.VMEM((ACH,), jnp.int32),
        pltpu.VMEM((ACH,), jnp.int32),
        pltpu.VMEM((NBPAD,), jnp.int32),
        pltpu.VMEM((16,), jnp.int32),
        pltpu.VMEM((ACH, H), jnp.float32),
        pltpu.SemaphoreType.DMA,
    ],
)(_dispatch_kernel)

_combine = functools.partial(
    pl.kernel,
    compiler_params=_sc_params,
    out_type=jax.ShapeDtypeStruct((S, H), jnp.float32),
    mesh=_sc_mesh,
    scratch_types=[
        pltpu.VMEM((TCH,), jnp.int32),
        pltpu.VMEM((TCH,), jnp.int32),
        pltpu.VMEM((TCH,), jnp.float32),
        pltpu.VMEM((TCH,), jnp.float32),
        pltpu.VMEM((TCH, H), jnp.float32),
        pltpu.VMEM((TCH, H), jnp.float32),
        pltpu.SemaphoreType.DMA,
    ],
)(_combine_kernel)


@jax.jit
def _forward(x, task_embeddings, params):
    x2d = x.reshape(S, H)
    te2d = task_embeddings.reshape(S, AD)

    (pm, ent_part, i1, i2, p1, p2, c1, c2, ap_all, cgi_all) = pl.pallas_call(
        _router_kernel,
        grid=(NTB_A,),
        in_specs=[
            pl.BlockSpec((TB_A, H), lambda i: (i, 0)),
            pl.BlockSpec((TB_A, AD), lambda i: (i, 0)),
            pl.BlockSpec((H + AD, HP), lambda i: (0, 0)),
            pl.BlockSpec((1, HP), lambda i: (0, 0)),
            pl.BlockSpec((HP, H), lambda i: (0, 0)),
            pl.BlockSpec((1, H), lambda i: (0, 0)),
            pl.BlockSpec((H, E), lambda i: (0, 0)),
            pl.BlockSpec((1, E), lambda i: (0, 0)),
            pl.BlockSpec((TD, E), lambda i: (0, 0)),
            pl.BlockSpec((1, 1, AD), lambda i: (i, 0, 0)),
            pl.BlockSpec((1, AD, H), lambda i: (i, 0, 0)),
            pl.BlockSpec((1, 1, H), lambda i: (i, 0, 0)),
            pl.BlockSpec((1, H, I), lambda i: (i, 1, 0)),
            pl.BlockSpec((1, 1, I), lambda i: (i, 0, 0)),
        ],
        out_specs=[
            pl.BlockSpec((TB_A, E), lambda i: (i, 0)),
            pl.BlockSpec((1, 8, 128), lambda i: (i, 0, 0)),
            pl.BlockSpec((TB_A, 1), lambda i: (i, 0)),
            pl.BlockSpec((TB_A, 1), lambda i: (i, 0)),
            pl.BlockSpec((TB_A, 1), lambda i: (i, 0)),
            pl.BlockSpec((TB_A, 1), lambda i: (i, 0)),
            pl.BlockSpec((1, 2, 16), lambda i: (i, 0, 0)),
            pl.BlockSpec((1, 2, 16), lambda i: (i, 0, 0)),
            pl.BlockSpec((1, 1, H), lambda i: (i, 0, 0)),
            pl.BlockSpec((1, 1, I), lambda i: (i, 0, 0)),
        ],
        out_shape=[
            jax.ShapeDtypeStruct((S, E), jnp.float32),
            jax.ShapeDtypeStruct((NTB_A, 8, 128), jnp.float32),
            jax.ShapeDtypeStruct((S, 1), jnp.int32),
            jax.ShapeDtypeStruct((S, 1), jnp.int32),
            jax.ShapeDtypeStruct((S, 1), jnp.float32),
            jax.ShapeDtypeStruct((S, 1), jnp.float32),
            jax.ShapeDtypeStruct((NTB_A, 2, 16), jnp.int32),
            jax.ShapeDtypeStruct((NTB_A, 2, 16), jnp.int32),
            jax.ShapeDtypeStruct((E, 1, H), jnp.float32),
            jax.ShapeDtypeStruct((E, 1, I), jnp.float32),
        ],
    )(x2d, te2d, params['Wip'], params['bip'].reshape(1, HP),
      params['Wmp'], params['bmp'].reshape(1, H),
      params['Wr'], params['br'].reshape(1, E), params['P_attr'],
      params['attr_emb'].reshape(E, 1, AD), params['Wap'],
      params['bap'].reshape(E, 1, H), params['Wg1'],
      params['bg1'].reshape(E, 1, I))

    xd, dpos, be, used = _dispatch(i1.reshape(S), i2.reshape(S),
                                   c1.reshape(NW * 8), c2.reshape(NW * 8),
                                   x2d)

    yd = pl.pallas_call(
        _group_ffn_kernel,
        grid_spec=pltpu.PrefetchScalarGridSpec(
            num_scalar_prefetch=2,
            grid=(NB,),
            in_specs=[
                pl.BlockSpec((1, 1, H), lambda b, be, u: (be[b], 0, 0)),
                pl.BlockSpec((1, 1, I), lambda b, be, u: (be[b], 0, 0)),
                pl.BlockSpec((1, H, I), lambda b, be, u: (be[b], 0, 0)),
                pl.BlockSpec((1, I, H), lambda b, be, u: (be[b], 0, 0)),
                pl.BlockSpec((1, 1, H), lambda b, be, u: (be[b], 0, 0)),
                pl.BlockSpec((1, H, I), lambda b, be, u: (be[b], 0, 0)),
                pl.BlockSpec((1, 1, I), lambda b, be, u: (be[b], 0, 0)),
                pl.BlockSpec((1, I, H), lambda b, be, u: (be[b], 0, 0)),
                pl.BlockSpec((1, 1, H), lambda b, be, u: (be[b], 0, 0)),
                pl.BlockSpec((1, 1, H), lambda b, be, u: (be[b], 0, 0)),
                pl.BlockSpec((1, 1, H), lambda b, be, u: (be[b], 0, 0)),
                pl.BlockSpec((BLKS, H), lambda b, be, u: (b, 0)),
            ],
            out_specs=pl.BlockSpec((BLKS, H), lambda b, be, u: (b, 0)),
        ),
        out_shape=jax.ShapeDtypeStruct((C, H), jnp.float32),
    )(be, used, ap_all, cgi_all, params['Wg1'],
      params['Wg2'], params['bg2'].reshape(E, 1, H),
      params['Wf1'], params['bf1'].reshape(E, 1, I),
      params['Wf2'], params['bf2'].reshape(E, 1, H),
      params['ln_g'].reshape(E, 1, H), params['ln_b'].reshape(E, 1, H),
      xd)

    out = _combine(yd, dpos, p1[:, 0], p2[:, 0])

    entropy_loss = -(jnp.sum(ent_part[:, 0, 0]) / S)
    return out.reshape(x.shape), entropy_loss


def kernel(x, task_embeddings, params):
    return _forward(x, task_embeddings, params)


# speedup vs baseline: 1.1803x; 1.0062x over previous
import functools

import jax
import jax.numpy as jnp
from jax import lax
from jax.experimental import pallas as pl
from jax.experimental.pallas import tpu as pltpu
from jax.experimental.pallas import tpu_sc as plsc

H = 768
E = 8
T = 4
TD = 64
K = 2
I = 1536
S = 2048
AD = TD * T
HP = 4 * H

TB_A = 256
NTB_A = S // TB_A

NA = K * S
BLKS = 512
NB = NA // BLKS + E
C = NB * BLKS

NBPAD = ((NB + 15) // 16) * 16

NC = 2
NS = 16
NW = NC * NS
ACH = NA // NW
TCH = S // NW


def _sigmoid(x):
    return 1.0 / (1.0 + jnp.exp(-x))


def _router_kernel(x_ref, te_ref, wip_ref, bip_ref, wmp_ref, bmp_ref,
                   wr_ref, br_ref, pattr_ref, attr_ref, wap_ref, bap_ref,
                   wg1a_ref, bg1_ref, pm_ref, ent_ref,
                   i1_ref, i2_ref, p1_ref, p2_ref, c1_ref, c2_ref,
                   apo_ref, cgi_ref):
    ap = attr_ref[0] @ wap_ref[0] + bap_ref[0]
    apo_ref[0] = ap
    cgi_ref[0] = ap @ wg1a_ref[0] + bg1_ref[0]
    xs = x_ref[...]
    te = te_ref[...]
    h = xs @ wip_ref[:H, :] + te @ wip_ref[H:, :] + bip_ref[...]
    h = jnp.maximum(h, 0.0)
    h = h @ wmp_ref[...] + bmp_ref[...]
    h = jnp.maximum(h, 0.0)
    logits = h @ wr_ref[...] + br_ref[...]
    logits = logits - jnp.max(logits, axis=-1, keepdims=True)
    el = jnp.exp(logits)
    ep = el / jnp.sum(el, axis=-1, keepdims=True)
    acc = jnp.zeros((TB_A, E), jnp.float32)
    for t in range(T):
        asc = te[:, t * TD:(t + 1) * TD] @ pattr_ref[...]
        asc = asc - jnp.max(asc, axis=-1, keepdims=True)
        ea = jnp.exp(asc)
        acc = acc + ea / jnp.sum(ea, axis=-1, keepdims=True)
    p = ep * (acc * (1.0 / T))
    iota8 = lax.broadcasted_iota(jnp.int32, (TB_A, E), 1)
    m1 = jnp.max(p, axis=-1, keepdims=True)
    i1 = jnp.min(jnp.where(p == m1, iota8, E), axis=-1, keepdims=True)
    px = jnp.where(iota8 == i1, -1.0, p)
    m2 = jnp.max(px, axis=-1, keepdims=True)
    i2 = jnp.min(jnp.where(px == m2, iota8, E), axis=-1, keepdims=True)
    mask1 = iota8 == i1
    mask2 = iota8 == i2
    pm = p * jnp.logical_or(mask1, mask2).astype(jnp.float32)
    pm_ref[...] = pm
    ent = jnp.sum(pm * jnp.log(pm + 1e-08))
    ent_ref[...] = jnp.full((1, 8, 128), ent, jnp.float32)
    i1_ref[...] = i1
    i2_ref[...] = i2
    p1_ref[...] = m1
    p2_ref[...] = m2
    m1i = mask1.astype(jnp.int32)
    m2i = mask2.astype(jnp.int32)
    z8 = jnp.zeros((1, 8), jnp.int32)
    c1_ref[0] = jnp.concatenate(
        [jnp.sum(m1i[:128], axis=0, keepdims=True), z8,
         jnp.sum(m1i[128:], axis=0, keepdims=True), z8], axis=1).reshape(2, 16)
    c2_ref[0] = jnp.concatenate(
        [jnp.sum(m2i[:128], axis=0, keepdims=True), z8,
         jnp.sum(m2i[128:], axis=0, keepdims=True), z8], axis=1).reshape(2, 16)


def _dispatch_kernel(i1_hbm, i2_hbm, c1_hbm, c2_hbm, x_hbm,
                     xd_hbm, dpos_hbm, be_hbm, used_hbm,
                     ids_v, cnts_v, tbuf_v, dbuf_v, bebuf_v, used_v,
                     rows_v, sem):
    wid = lax.axis_index("s") * NC + lax.axis_index("c")
    iota = jnp.arange(16, dtype=jnp.int32)

    @pl.when(wid < NW // 2)
    def _():
        pltpu.sync_copy(i1_hbm.at[pl.ds(wid * ACH, ACH)], ids_v)

    @pl.when(wid >= NW // 2)
    def _():
        pltpu.sync_copy(i2_hbm.at[pl.ds(wid * ACH - S, ACH)], ids_v)

    pltpu.sync_copy(c1_hbm, cnts_v.at[pl.ds(0, NW * 8)])
    pltpu.sync_copy(c2_hbm, cnts_v.at[pl.ds(NW * 8, NW * 8)])

    def crow(r, carry):
        tot, pre = carry
        row = cnts_v[pl.ds(r * 16, 16)]
        pre = jnp.where(r < wid, pre + row, pre)
        tot = tot + row
        return tot, pre

    zero16 = jnp.zeros(16, jnp.int32)
    tot, pre = lax.fori_loop(0, NW, crow, (zero16, zero16))

    npad = jnp.where(iota < E, (tot + (BLKS - 1)) & (-BLKS), 0)
    ends = plsc.cumsum(npad)
    breg = ends - npad + pre

    for j in range(ACH // 16):
        ids = ids_v[pl.ds(j * 16, 16)]
        av = iota + (wid * ACH + j * 16)
        tbuf_v[pl.ds(j * 16, 16)] = av & (S - 1)
        dest = zero16
        for e in range(E):
            m = ids == e
            r = plsc.cumsum(m.astype(jnp.int32))
            bsp = jnp.sum(jnp.where(iota == e, breg, 0))
            dest = jnp.where(m, r - 1 + bsp, dest)
            pc = plsc.all_reduce_population_count(m)
            breg = breg + jnp.where(iota == e, pc, 0)
        dbuf_v[pl.ds(j * 16, 16)] = dest

    cp_g = pltpu.async_copy(x_hbm.at[tbuf_v], rows_v, sem)
    cp_g.wait()
    cp_s = pltpu.async_copy(rows_v, xd_hbm.at[dbuf_v], sem)
    cp_s.wait()
    pltpu.sync_copy(dbuf_v, dpos_hbm.at[pl.ds(wid * ACH, ACH)])

    @pl.when(wid == 0)
    def _():
        for vi in range(NBPAD // 16):
            bv = (iota + vi * 16) * BLKS
            acc = jnp.zeros(16, jnp.int32)
            for e in range(E):
                ee = jnp.sum(jnp.where(iota == e, ends, 0))
                acc = acc + (bv >= ee).astype(jnp.int32)
            bebuf_v[pl.ds(vi * 16, 16)] = jnp.minimum(acc, E - 1)
        pltpu.sync_copy(bebuf_v, be_hbm)
        used_v[...] = jnp.sum(jnp.where(iota == E - 1, ends, 0)) + iota * 0
        pltpu.sync_copy(used_v, used_hbm)


def _group_ffn_kernel(be_ref, used_ref, ap_ref, cgi_ref, wg1x_ref, wg2_ref,
                      bg2_ref, wf1_ref, bf1_ref, wf2_ref, bf2_ref, lng_ref,
                      lnb_ref, xd_ref, yd_ref):
    b = pl.program_id(0)

    @pl.when(b * BLKS < used_ref[0])
    def _():
        _group_ffn_body(ap_ref, cgi_ref, wg1x_ref, wg2_ref, bg2_ref,
                        wf1_ref, bf1_ref, wf2_ref, bf2_ref, lng_ref,
                        lnb_ref, xd_ref, yd_ref)


def _group_ffn_body(ap_ref, cgi_ref, wg1x_ref, wg2_ref, bg2_ref,
                    wf1_ref, bf1_ref, wf2_ref, bf2_ref, lng_ref,
                    lnb_ref, xd_ref, yd_ref):
    ap = ap_ref[0]
    c_gi = cgi_ref[0]
    xs = xd_ref[...]
    gi = xs @ wg1x_ref[0] + c_gi
    g = jnp.maximum(gi, 0.0) @ wg2_ref[0] + bg2_ref[0]
    g = _sigmoid(g)
    x2 = xs * g + ap * (1.0 - g)
    x3 = jnp.maximum(x2 @ wf1_ref[0] + bf1_ref[0], 0.0)
    y = x3 @ wf2_ref[0] + bf2_ref[0] + xs
    m = jnp.mean(y, axis=-1, keepdims=True)
    yc = y - m
    v = jnp.mean(yc * yc, axis=-1, keepdims=True)
    yd_ref[...] = yc / jnp.sqrt(v + 1e-05) * lng_ref[0] + lnb_ref[0]


def _combine_kernel(yd_hbm, dpos_hbm, p1_hbm, p2_hbm, out_hbm,
                    idx0_v, idx1_v, p1_v, p2_v, buf0_v, buf1_v, sem):
    wid = lax.axis_index("s") * NC + lax.axis_index("c")
    base_t = wid * TCH
    pltpu.sync_copy(dpos_hbm.at[pl.ds(base_t, TCH)], idx0_v)
    pltpu.sync_copy(dpos_hbm.at[pl.ds(S + base_t, TCH)], idx1_v)
    pltpu.sync_copy(p1_hbm.at[pl.ds(base_t, TCH)], p1_v)
    pltpu.sync_copy(p2_hbm.at[pl.ds(base_t, TCH)], p2_v)
    cp0 = pltpu.async_copy(yd_hbm.at[idx0_v], buf0_v, sem)
    cp1 = pltpu.async_copy(yd_hbm.at[idx1_v], buf1_v, sem)
    cp0.wait()
    cp1.wait()

    def row(i, _):
        fi = jnp.full((16,), i, jnp.int32)
        p1b = plsc.load_gather(p1_v, [fi])
        p2b = plsc.load_gather(p2_v, [fi])
        for ch in range(H // 16):
            sl = pl.ds(ch * 16, 16)
            buf0_v[i, sl] = buf0_v[i, sl] * p1b + buf1_v[i, sl] * p2b
        return 0

    lax.fori_loop(0, TCH, row, 0)
    pltpu.sync_copy(buf0_v, out_hbm.at[pl.ds(base_t, TCH)])


_sc_mesh = plsc.VectorSubcoreMesh(
    core_axis_name="c", subcore_axis_name="s", num_cores=NC, num_subcores=NS)

_sc_params = pltpu.CompilerParams(needs_layout_passes=False)

_dispatch = functools.partial(
    pl.kernel,
    compiler_params=_sc_params,
    out_type=[
        jax.ShapeDtypeStruct((C, H), jnp.float32),
        jax.ShapeDtypeStruct((NA,), jnp.int32),
        jax.ShapeDtypeStruct((NBPAD,), jnp.int32),
        jax.ShapeDtypeStruct((16,), jnp.int32),
    ],
    mesh=_sc_mesh,
    scratch_types=[
        pltpu.VMEM((ACH,), jnp.int32),
        pltpu.VMEM((NW * 16,), jnp.int32),
        pltpu.VMEM((ACH,), jnp.int32),
        pltpu.VMEM((ACH,), jnp.int32),
        pltpu.VMEM((NBPAD,), jnp.int32),
        pltpu.VMEM((16,), jnp.int32),
        pltpu.VMEM((ACH, H), jnp.float32),
        pltpu.SemaphoreType.DMA,
    ],
)(_dispatch_kernel)

_combine = functools.partial(
    pl.kernel,
    compiler_params=_sc_params,
    out_type=jax.ShapeDtypeStruct((S, H), jnp.float32),
    mesh=_sc_mesh,
    scratch_types=[
        pltpu.VMEM((TCH,), jnp.int32),
        pltpu.VMEM((TCH,), jnp.int32),
        pltpu.VMEM((TCH,), jnp.float32),
        pltpu.VMEM((TCH,), jnp.float32),
        pltpu.VMEM((TCH, H), jnp.float32),
        pltpu.VMEM((TCH, H), jnp.float32),
        pltpu.SemaphoreType.DMA,
    ],
)(_combine_kernel)


@jax.jit
def _forward(x, task_embeddings, params):
    x2d = x.reshape(S, H)
    te2d = task_embeddings.reshape(S, AD)

    (pm, ent_part, i1, i2, p1, p2, c1, c2, ap_all, cgi_all) = pl.pallas_call(
        _router_kernel,
        grid=(NTB_A,),
        in_specs=[
            pl.BlockSpec((TB_A, H), lambda i: (i, 0)),
            pl.BlockSpec((TB_A, AD), lambda i: (i, 0)),
            pl.BlockSpec((H + AD, HP), lambda i: (0, 0)),
            pl.BlockSpec((1, HP), lambda i: (0, 0)),
            pl.BlockSpec((HP, H), lambda i: (0, 0)),
            pl.BlockSpec((1, H), lambda i: (0, 0)),
            pl.BlockSpec((H, E), lambda i: (0, 0)),
            pl.BlockSpec((1, E), lambda i: (0, 0)),
            pl.BlockSpec((TD, E), lambda i: (0, 0)),
            pl.BlockSpec((1, 1, AD), lambda i: (i, 0, 0)),
            pl.BlockSpec((1, AD, H), lambda i: (i, 0, 0)),
            pl.BlockSpec((1, 1, H), lambda i: (i, 0, 0)),
            pl.BlockSpec((1, H, I), lambda i: (i, 1, 0)),
            pl.BlockSpec((1, 1, I), lambda i: (i, 0, 0)),
        ],
        out_specs=[
            pl.BlockSpec((TB_A, E), lambda i: (i, 0)),
            pl.BlockSpec((1, 8, 128), lambda i: (i, 0, 0)),
            pl.BlockSpec((TB_A, 1), lambda i: (i, 0)),
            pl.BlockSpec((TB_A, 1), lambda i: (i, 0)),
            pl.BlockSpec((TB_A, 1), lambda i: (i, 0)),
            pl.BlockSpec((TB_A, 1), lambda i: (i, 0)),
            pl.BlockSpec((1, 2, 16), lambda i: (i, 0, 0)),
            pl.BlockSpec((1, 2, 16), lambda i: (i, 0, 0)),
            pl.BlockSpec((1, 1, H), lambda i: (i, 0, 0)),
            pl.BlockSpec((1, 1, I), lambda i: (i, 0, 0)),
        ],
        out_shape=[
            jax.ShapeDtypeStruct((S, E), jnp.float32),
            jax.ShapeDtypeStruct((NTB_A, 8, 128), jnp.float32),
            jax.ShapeDtypeStruct((S, 1), jnp.int32),
            jax.ShapeDtypeStruct((S, 1), jnp.int32),
            jax.ShapeDtypeStruct((S, 1), jnp.float32),
            jax.ShapeDtypeStruct((S, 1), jnp.float32),
            jax.ShapeDtypeStruct((NTB_A, 2, 16), jnp.int32),
            jax.ShapeDtypeStruct((NTB_A, 2, 16), jnp.int32),
            jax.ShapeDtypeStruct((E, 1, H), jnp.float32),
            jax.ShapeDtypeStruct((E, 1, I), jnp.float32),
        ],
    )(x2d, te2d, params['Wip'], params['bip'].reshape(1, HP),
      params['Wmp'], params['bmp'].reshape(1, H),
      params['Wr'], params['br'].reshape(1, E), params['P_attr'],
      params['attr_emb'].reshape(E, 1, AD), params['Wap'],
      params['bap'].reshape(E, 1, H), params['Wg1'],
      params['bg1'].reshape(E, 1, I))

    xd, dpos, be, used = _dispatch(i1.reshape(S), i2.reshape(S),
                                   c1.reshape(NW * 8), c2.reshape(NW * 8),
                                   x2d)

    yd = pl.pallas_call(
        _group_ffn_kernel,
        grid_spec=pltpu.PrefetchScalarGridSpec(
            num_scalar_prefetch=2,
            grid=(NB,),
            in_specs=[
                pl.BlockSpec((1, 1, H), lambda b, be, u: (be[b], 0, 0)),
                pl.BlockSpec((1, 1, I), lambda b, be, u: (be[b], 0, 0)),
                pl.BlockSpec((1, H, I), lambda b, be, u: (be[b], 0, 0)),
                pl.BlockSpec((1, I, H), lambda b, be, u: (be[b], 0, 0)),
                pl.BlockSpec((1, 1, H), lambda b, be, u: (be[b], 0, 0)),
                pl.BlockSpec((1, H, I), lambda b, be, u: (be[b], 0, 0)),
                pl.BlockSpec((1, 1, I), lambda b, be, u: (be[b], 0, 0)),
                pl.BlockSpec((1, I, H), lambda b, be, u: (be[b], 0, 0)),
                pl.BlockSpec((1, 1, H), lambda b, be, u: (be[b], 0, 0)),
                pl.BlockSpec((1, 1, H), lambda b, be, u: (be[b], 0, 0)),
                pl.BlockSpec((1, 1, H), lambda b, be, u: (be[b], 0, 0)),
                pl.BlockSpec(
                    (BLKS, H),
                    lambda b, be, u: (jnp.where(b * BLKS < u[0], b, 0), 0)),
            ],
            out_specs=pl.BlockSpec((BLKS, H), lambda b, be, u: (b, 0)),
        ),
        out_shape=jax.ShapeDtypeStruct((C, H), jnp.float32),
    )(be, used, ap_all, cgi_all, params['Wg1'],
      params['Wg2'], params['bg2'].reshape(E, 1, H),
      params['Wf1'], params['bf1'].reshape(E, 1, I),
      params['Wf2'], params['bf2'].reshape(E, 1, H),
      params['ln_g'].reshape(E, 1, H), params['ln_b'].reshape(E, 1, H),
      xd)

    out = _combine(yd, dpos, p1[:, 0], p2[:, 0])

    entropy_loss = -(jnp.sum(ent_part[:, 0, 0]) / S)
    return out.reshape(x.shape), entropy_loss


def kernel(x, task_embeddings, params):
    return _forward(x, task_embeddings, params)
